# baseline (speedup 1.0000x reference)
import functools

import jax
import jax.numpy as jnp
from jax import lax
from jax.experimental import pallas as pl
from jax.experimental.pallas import tpu as pltpu
from jax.experimental.pallas import tpu_sc as plsc

ROWS = 256
OPAD = 16
CHUNK = 128


PARTS = 8
IDB = 3
SORT_NET = {
    4: ((0, 1), (2, 3), (0, 2), (1, 3), (1, 2)),
    8: ((0, 1), (2, 3), (4, 5), (6, 7),
        (0, 2), (1, 3), (4, 6), (5, 7),
        (1, 2), (5, 6),
        (0, 4), (1, 5), (2, 6), (3, 7),
        (2, 4), (3, 5),
        (1, 2), (3, 4), (5, 6)),
}


def _topk_body(n, k, base, scale_ref, x_rows_ref, x_full_ref, w_ref,
               bias_ref, bond_ref, gidx_ref, stack_ref, p2_ref, oc_ref,
               d_ref):
    i = pl.program_id(1)
    q = n // PARTS
    shq = q.bit_length() - 1
    w = w_ref[...]
    bias = bias_ref[...]

    def feat(v):
        t = lax.dot_general(v, w, (((1,), (1,)), ((), ())),
                            preferred_element_type=jnp.float32) + bias
        return jnp.where(t > 0, t, 0.01 * t)

    xt_rows = feat(x_rows_ref[0])
    xt_full = feat(x_full_ref[0])
    sim = lax.dot_general(xt_rows, xt_full, (((1,), (1,)), ((), ())),
                          preferred_element_type=jnp.float32)
    sim = sim * scale_ref[...]
    rows = i * ROWS + lax.broadcasted_iota(jnp.int32, (ROWS, n), 0)
    cols = lax.broadcasted_iota(jnp.int32, (ROWS, n), 1)
    sim = jnp.where(cols == rows, -jnp.inf, sim)

    parts = [lax.slice(sim, (0, m * q), (ROWS, (m + 1) * q))
             for m in range(PARTS)]
    ids = [jnp.full((ROWS, q), m, jnp.int32) for m in range(PARTS)]

    def cswap(a, ia, c, ic):
        sw = (c > a) | ((c == a) & (ic < ia))
        return (jnp.where(sw, c, a), jnp.where(sw, ic, ia),
                jnp.where(sw, a, c), jnp.where(sw, ia, ic))

    for x_, y_ in SORT_NET[PARTS]:
        parts[x_], ids[x_], parts[y_], ids[y_] = cswap(
            parts[x_], ids[x_], parts[y_], ids[y_])
    lane = lax.broadcasted_iota(jnp.int32, (ROWS, q), 1)
    for m in range(PARTS):
        stack_ref[:, m * q:(m + 1) * q] = parts[m]
    packed = ids[0]
    for m in range(1, PARTS):
        packed = packed | (ids[m] << (IDB * m))
    p2_ref[...] = packed
    oc_ref[...] = (ids[0] << shq) | lane
    d_ref[...] = jnp.zeros((ROWS, q), jnp.int32)

    kcols = lax.broadcasted_iota(jnp.int32, (ROWS, k), 1)

    def step(j, carry):
        vals, idxs = carry
        head = stack_ref[:, :q]
        ocv = oc_ref[...]
        m = jnp.max(head, axis=1, keepdims=True)
        eq = head == m
        outcol = jnp.min(jnp.where(eq, ocv, n), axis=1, keepdims=True)
        eqam = eq & (ocv == outcol)
        dn = d_ref[...] + 1
        nxt = jnp.full((ROWS, q), -jnp.inf)
        for lv in range(PARTS - 1, 0, -1):
            nxt = jnp.where(dn == lv, stack_ref[:, lv * q:(lv + 1) * q],
                            nxt)
        newmem = (p2_ref[...] >> (dn * IDB)) & (PARTS - 1)
        stack_ref[:, :q] = jnp.where(eqam, nxt, head)
        oc_ref[...] = jnp.where(eqam, (newmem << shq) | lane, ocv)
        d_ref[...] = jnp.where(eqam, dn, dn - 1)
        vals = jnp.where(kcols == j, m, vals)
        idxs = jnp.where(kcols == j, outcol, idxs)
        return vals, idxs

    vals0 = jnp.zeros((ROWS, k), jnp.float32)
    idxs0 = jnp.zeros((ROWS, k), jnp.int32)
    vals, idxs = lax.fori_loop(0, k, step, (vals0, idxs0))
    bond_ref[0] = 1.0 / (1.0 + jnp.exp(-vals))
    gidx_ref[0] = idxs + (base + pl.program_id(0) * n)


def _topk_call(x, w_pad, bias_pad, scale, k, base):
    bsz, n, d = x.shape
    grid = (bsz, n // ROWS)
    return pl.pallas_call(
        functools.partial(_topk_body, n, k, base),
        grid=grid,
        in_specs=[
            pl.BlockSpec((1, 1), lambda b, i: (0, 0)),
            pl.BlockSpec((1, ROWS, d), lambda b, i: (b, i, 0)),
            pl.BlockSpec((1, n, d), lambda b, i: (b, 0, 0)),
            pl.BlockSpec((OPAD, d), lambda b, i: (0, 0)),
            pl.BlockSpec((1, OPAD), lambda b, i: (0, 0)),
        ],
        out_specs=[
            pl.BlockSpec((1, ROWS, k), lambda b, i: (b, i, 0)),
            pl.BlockSpec((1, ROWS, k), lambda b, i: (b, i, 0)),
        ],
        out_shape=[
            jax.ShapeDtypeStruct((bsz, n, k), jnp.float32),
            jax.ShapeDtypeStruct((bsz, n, k), jnp.int32),
        ],
        scratch_shapes=[pltpu.VMEM((ROWS, n), jnp.float32),
                        pltpu.VMEM((ROWS, n // PARTS), jnp.int32),
                        pltpu.VMEM((ROWS, n // PARTS), jnp.int32),
                        pltpu.VMEM((ROWS, n // PARTS), jnp.int32)],
    )(scale, x, x, w_pad, bias_pad)


def _gather_call(table, flat_idx):
    total, d = table.shape[0], table.shape[1]
    g = flat_idx.shape[0]
    info = plsc.get_sparse_core_info()
    nw = info.num_cores * info.num_subcores
    per_w = g // nw
    mesh = plsc.VectorSubcoreMesh(core_axis_name="c", subcore_axis_name="s")

    nchunks = per_w // CHUNK
    nbuf = 4

    @functools.partial(
        pl.kernel,
        out_type=jax.ShapeDtypeStruct((g, d), jnp.float32),
        mesh=mesh,
        scratch_types=[
            pltpu.VMEM((nbuf, CHUNK), jnp.int32),
            pltpu.VMEM((nbuf, CHUNK, d), jnp.float32),
            pltpu.SemaphoreType.DMA((nbuf,)),
            pltpu.SemaphoreType.DMA((nbuf,)),
            pltpu.SemaphoreType.DMA((nbuf,)),
        ],
    )
    def gather_k(table_hbm, idx_hbm, out_hbm, idx_v, rows_v, isem, gsem,
                 ssem):
        wid = lax.axis_index("s") * info.num_cores + lax.axis_index("c")
        base = wid * per_w

        def idx_cp(j, s):
            return pltpu.make_async_copy(
                idx_hbm.at[pl.ds(base + j * CHUNK, CHUNK)], idx_v.at[s],
                isem.at[s])

        def gather_cp(s):
            return pltpu.make_async_copy(table_hbm.at[idx_v.at[s]],
                                         rows_v.at[s], gsem.at[s])

        def scatter_cp(j, s):
            return pltpu.make_async_copy(
                rows_v.at[s], out_hbm.at[pl.ds(base + j * CHUNK, CHUNK)],
                ssem.at[s])

        idx_cp(0, 0).start()
        idx_cp(1, 1).start()

        def body(j, carry):
            s = j % nbuf
            idx_cp(j, s).wait()

            @pl.when(j >= nbuf)
            def _():
                scatter_cp(j - nbuf, s).wait()

            gather_cp(s).start()

            @pl.when(j >= 2)
            def _():
                gather_cp((j - 2) % nbuf).wait()
                scatter_cp(j - 2, (j - 2) % nbuf).start()

            @pl.when(j + 2 < nchunks)
            def _():
                idx_cp(j + 2, (j + 2) % nbuf).start()

            return carry

        lax.fori_loop(0, nchunks, body, 0)
        for t in (nchunks - 2, nchunks - 1):
            gather_cp(t % nbuf).wait()
            scatter_cp(t, t % nbuf).start()
        for t in range(nchunks - nbuf, nchunks):
            scatter_cp(t, t % nbuf).wait()

    return gather_k(table, flat_idx)


def kernel(x, edge_weight, node_weight, node_bias):
    bsz, n, d = x.shape
    k = 32
    o = node_weight.shape[0]
    w_pad = jnp.zeros((OPAD, d), jnp.float32).at[:o].set(node_weight)
    bias_pad = jnp.zeros((1, OPAD), jnp.float32).at[0, :o].set(node_bias)
    scale = jnp.exp(edge_weight).reshape(1, 1).astype(jnp.float32)

    table = x.reshape(bsz * n, d)
    h = bsz // 2
    bonds, rows = [], []
    for b in range(0, bsz, h):
        bond_b, gidx_b = _topk_call(x[b:b + h], w_pad, bias_pad, scale, k,
                                    b * n)
        bonds.append(bond_b)
        rows.append(_gather_call(table, gidx_b.reshape(h * n * k)))
    node_neighbor = jnp.concatenate(rows).reshape(bsz, n, k, d)
    bond_neighbor = jnp.concatenate(bonds).reshape(bsz, n, k, 1)
    return node_neighbor, bond_neighbor


# speedup vs baseline: 1.0675x; 1.0675x over previous
import functools

import jax
import jax.numpy as jnp
from jax import lax
from jax.experimental import pallas as pl
from jax.experimental.pallas import tpu as pltpu
from jax.experimental.pallas import tpu_sc as plsc

ROWS = 256
OPAD = 16
CHUNK = 128


PARTS = 8
IDB = 3
SORT_NET = {
    4: ((0, 1), (2, 3), (0, 2), (1, 3), (1, 2)),
    8: ((0, 1), (2, 3), (4, 5), (6, 7),
        (0, 2), (1, 3), (4, 6), (5, 7),
        (1, 2), (5, 6),
        (0, 4), (1, 5), (2, 6), (3, 7),
        (2, 4), (3, 5),
        (1, 2), (3, 4), (5, 6)),
}


def _topk_body(n, k, base, scale_ref, x_rows_ref, x_full_ref, w_ref,
               bias_ref, bond_ref, gidx_ref, stack_ref, p2_ref, oc_ref,
               d_ref):
    i = pl.program_id(1)
    q = n // PARTS
    shq = q.bit_length() - 1
    w = w_ref[...]
    bias = bias_ref[...]

    def feat(v):
        t = lax.dot_general(v, w, (((1,), (1,)), ((), ())),
                            preferred_element_type=jnp.float32) + bias
        return jnp.where(t > 0, t, 0.01 * t)

    xt_rows = feat(x_rows_ref[0])
    xt_full = feat(x_full_ref[0])
    sim = lax.dot_general(xt_rows, xt_full, (((1,), (1,)), ((), ())),
                          preferred_element_type=jnp.float32)
    sim = sim * scale_ref[...]
    rows = i * ROWS + lax.broadcasted_iota(jnp.int32, (ROWS, n), 0)
    cols = lax.broadcasted_iota(jnp.int32, (ROWS, n), 1)
    sim = jnp.where(cols == rows, -jnp.inf, sim)

    parts = [lax.slice(sim, (0, m * q), (ROWS, (m + 1) * q))
             for m in range(PARTS)]
    ids = [jnp.full((ROWS, q), m, jnp.int32) for m in range(PARTS)]

    def cswap(a, ia, c, ic):
        sw = (c > a) | ((c == a) & (ic < ia))
        return (jnp.where(sw, c, a), jnp.where(sw, ic, ia),
                jnp.where(sw, a, c), jnp.where(sw, ia, ic))

    for x_, y_ in SORT_NET[PARTS]:
        parts[x_], ids[x_], parts[y_], ids[y_] = cswap(
            parts[x_], ids[x_], parts[y_], ids[y_])
    lane = lax.broadcasted_iota(jnp.int32, (ROWS, q), 1)
    for m in range(PARTS):
        stack_ref[:, m * q:(m + 1) * q] = parts[m]
    packed = ids[0]
    for m in range(1, PARTS):
        packed = packed | (ids[m] << (IDB * m))
    p2_ref[...] = packed
    oc_ref[...] = (ids[0] << shq) | lane
    d_ref[...] = jnp.zeros((ROWS, q), jnp.int32)

    kcols = lax.broadcasted_iota(jnp.int32, (ROWS, k), 1)

    def step(j, carry):
        vals, idxs = carry
        head = stack_ref[:, :q]
        ocv = oc_ref[...]
        m = jnp.max(head, axis=1, keepdims=True)
        eq = head == m
        outcol = jnp.min(jnp.where(eq, ocv, n), axis=1, keepdims=True)
        eqam = eq & (ocv == outcol)
        dn = d_ref[...] + 1
        nxt = jnp.full((ROWS, q), -jnp.inf)
        for lv in range(PARTS - 1, 0, -1):
            nxt = jnp.where(dn == lv, stack_ref[:, lv * q:(lv + 1) * q],
                            nxt)
        newmem = (p2_ref[...] >> (dn * IDB)) & (PARTS - 1)
        stack_ref[:, :q] = jnp.where(eqam, nxt, head)
        oc_ref[...] = jnp.where(eqam, (newmem << shq) | lane, ocv)
        d_ref[...] = jnp.where(eqam, dn, dn - 1)
        vals = jnp.where(kcols == j, m, vals)
        idxs = jnp.where(kcols == j, outcol, idxs)
        return vals, idxs

    vals0 = jnp.zeros((ROWS, k), jnp.float32)
    idxs0 = jnp.zeros((ROWS, k), jnp.int32)
    vals, idxs = lax.fori_loop(0, k, step, (vals0, idxs0))
    bond_ref[0] = 1.0 / (1.0 + jnp.exp(-vals))
    gidx_ref[0] = idxs + (base + pl.program_id(0) * n)


def _topk_call(x, w_pad, bias_pad, scale, k, base):
    bsz, n, d = x.shape
    grid = (bsz, n // ROWS)
    return pl.pallas_call(
        functools.partial(_topk_body, n, k, base),
        grid=grid,
        in_specs=[
            pl.BlockSpec((1, 1), lambda b, i: (0, 0)),
            pl.BlockSpec((1, ROWS, d), lambda b, i: (b, i, 0)),
            pl.BlockSpec((1, n, d), lambda b, i: (b, 0, 0)),
            pl.BlockSpec((OPAD, d), lambda b, i: (0, 0)),
            pl.BlockSpec((1, OPAD), lambda b, i: (0, 0)),
        ],
        out_specs=[
            pl.BlockSpec((1, ROWS, k), lambda b, i: (b, i, 0)),
            pl.BlockSpec((1, ROWS, k), lambda b, i: (b, i, 0)),
        ],
        out_shape=[
            jax.ShapeDtypeStruct((bsz, n, k), jnp.float32),
            jax.ShapeDtypeStruct((bsz, n, k), jnp.int32),
        ],
        scratch_shapes=[pltpu.VMEM((ROWS, n), jnp.float32),
                        pltpu.VMEM((ROWS, n // PARTS), jnp.int32),
                        pltpu.VMEM((ROWS, n // PARTS), jnp.int32),
                        pltpu.VMEM((ROWS, n // PARTS), jnp.int32)],
    )(scale, x, x, w_pad, bias_pad)


def _gather_call(table, flat_idx):
    total, d = table.shape[0], table.shape[1]
    g = flat_idx.shape[0]
    info = plsc.get_sparse_core_info()
    nw = info.num_cores * info.num_subcores
    per_w = g // nw
    mesh = plsc.VectorSubcoreMesh(core_axis_name="c", subcore_axis_name="s")

    nchunks = per_w // CHUNK
    nbuf = 4

    @functools.partial(
        pl.kernel,
        out_type=jax.ShapeDtypeStruct((g, d), jnp.float32),
        mesh=mesh,
        scratch_types=[
            pltpu.VMEM((nbuf, CHUNK), jnp.int32),
            pltpu.VMEM((nbuf, CHUNK, d), jnp.float32),
            pltpu.SemaphoreType.DMA((nbuf,)),
            pltpu.SemaphoreType.DMA((nbuf,)),
            pltpu.SemaphoreType.DMA((nbuf,)),
        ],
    )
    def gather_k(table_hbm, idx_hbm, out_hbm, idx_v, rows_v, isem, gsem,
                 ssem):
        wid = lax.axis_index("s") * info.num_cores + lax.axis_index("c")
        base = wid * per_w

        def idx_cp(j, s):
            return pltpu.make_async_copy(
                idx_hbm.at[pl.ds(base + j * CHUNK, CHUNK)], idx_v.at[s],
                isem.at[s])

        def gather_cp(s):
            return pltpu.make_async_copy(table_hbm.at[idx_v.at[s]],
                                         rows_v.at[s], gsem.at[s])

        def scatter_cp(j, s):
            return pltpu.make_async_copy(
                rows_v.at[s], out_hbm.at[pl.ds(base + j * CHUNK, CHUNK)],
                ssem.at[s])

        idx_cp(0, 0).start()
        idx_cp(1, 1).start()

        def body(j, carry):
            s = j % nbuf
            idx_cp(j, s).wait()

            @pl.when(j >= nbuf)
            def _():
                scatter_cp(j - nbuf, s).wait()

            gather_cp(s).start()

            @pl.when(j >= 2)
            def _():
                gather_cp((j - 2) % nbuf).wait()
                scatter_cp(j - 2, (j - 2) % nbuf).start()

            @pl.when(j + 2 < nchunks)
            def _():
                idx_cp(j + 2, (j + 2) % nbuf).start()

            return carry

        lax.fori_loop(0, nchunks, body, 0)
        for t in (nchunks - 2, nchunks - 1):
            gather_cp(t % nbuf).wait()
            scatter_cp(t, t % nbuf).start()
        for t in range(nchunks - nbuf, nchunks):
            scatter_cp(t, t % nbuf).wait()

    return gather_k(table, flat_idx)


def kernel(x, edge_weight, node_weight, node_bias):
    bsz, n, d = x.shape
    k = 32
    o = node_weight.shape[0]
    w_pad = jnp.zeros((OPAD, d), jnp.float32).at[:o].set(node_weight)
    bias_pad = jnp.zeros((1, OPAD), jnp.float32).at[0, :o].set(node_bias)
    scale = jnp.exp(edge_weight).reshape(1, 1).astype(jnp.float32)

    table = x.reshape(bsz * n, d)
    bond, gidx = _topk_call(x, w_pad, bias_pad, scale, k, 0)
    rows = _gather_call(table, gidx.reshape(bsz * n * k))
    node_neighbor = rows.reshape(bsz, n, k, d)
    bond_neighbor = bond.reshape(bsz, n, k, 1)
    return node_neighbor, bond_neighbor


# speedup vs baseline: 1.1175x; 1.0469x over previous
import functools

import jax
import jax.numpy as jnp
from jax import lax
from jax.experimental import pallas as pl
from jax.experimental.pallas import tpu as pltpu
from jax.experimental.pallas import tpu_sc as plsc

ROWS = 256
OPAD = 16
CHUNK = 128


PARTS = 8
IDB = 3
SORT_NET = {
    4: ((0, 1), (2, 3), (0, 2), (1, 3), (1, 2)),
    8: ((0, 1), (2, 3), (4, 5), (6, 7),
        (0, 2), (1, 3), (4, 6), (5, 7),
        (1, 2), (5, 6),
        (0, 4), (1, 5), (2, 6), (3, 7),
        (2, 4), (3, 5),
        (1, 2), (3, 4), (5, 6)),
}


def _topk_body(n, k, base, scale_ref, x_rows_ref, x_full_ref, w_ref,
               bias_ref, bond_ref, gidx_ref, stack_ref, p2_ref, oc_ref,
               d_ref):
    i = pl.program_id(1)
    q = n // PARTS
    shq = q.bit_length() - 1
    w = w_ref[...]
    bias = bias_ref[...]

    def feat(v):
        t = lax.dot_general(v, w, (((1,), (1,)), ((), ())),
                            preferred_element_type=jnp.float32) + bias
        return jnp.where(t > 0, t, 0.01 * t)

    xt_rows = feat(x_rows_ref[0])
    xt_full = feat(x_full_ref[0])
    sim = lax.dot_general(xt_rows, xt_full, (((1,), (1,)), ((), ())),
                          preferred_element_type=jnp.float32)
    sim = sim * scale_ref[...]
    rows = i * ROWS + lax.broadcasted_iota(jnp.int32, (ROWS, n), 0)
    cols = lax.broadcasted_iota(jnp.int32, (ROWS, n), 1)
    sim = jnp.where(cols == rows, -jnp.inf, sim)

    parts = [lax.slice(sim, (0, m * q), (ROWS, (m + 1) * q))
             for m in range(PARTS)]
    ids = [jnp.full((ROWS, q), m, jnp.int32) for m in range(PARTS)]

    def cswap(a, ia, c, ic):
        sw = (c > a) | ((c == a) & (ic < ia))
        return (jnp.where(sw, c, a), jnp.where(sw, ic, ia),
                jnp.where(sw, a, c), jnp.where(sw, ia, ic))

    for x_, y_ in SORT_NET[PARTS]:
        parts[x_], ids[x_], parts[y_], ids[y_] = cswap(
            parts[x_], ids[x_], parts[y_], ids[y_])
    lane = lax.broadcasted_iota(jnp.int32, (ROWS, q), 1)
    for m in range(PARTS):
        stack_ref[:, m * q:(m + 1) * q] = parts[m]
    packed = ids[0]
    for m in range(1, PARTS):
        packed = packed | (ids[m] << (IDB * m))
    p2_ref[...] = packed
    oc_ref[...] = (ids[0] << shq) | lane
    d_ref[...] = jnp.zeros((ROWS, q), jnp.int32)

    kcols = lax.broadcasted_iota(jnp.int32, (ROWS, k), 1)

    def step(j, carry):
        vals, idxs = carry
        head = stack_ref[:, :q]
        ocv = oc_ref[...]
        m = jnp.max(head, axis=1, keepdims=True)
        eq = head == m
        outcol = jnp.min(jnp.where(eq, ocv, n), axis=1, keepdims=True)
        eqam = ocv == outcol
        d3o = d_ref[...]
        d3n = d3o + IDB
        nxt = jnp.full((ROWS, q), -jnp.inf)
        for lv in range(PARTS - 1, 0, -1):
            nxt = jnp.where(d3n == lv * IDB,
                            stack_ref[:, lv * q:(lv + 1) * q], nxt)
        newmem = (p2_ref[...] >> d3n) & (PARTS - 1)
        stack_ref[:, :q] = jnp.where(eqam, nxt, head)
        oc_ref[...] = jnp.where(eqam, (newmem << shq) | lane, ocv)
        d_ref[...] = jnp.where(eqam, d3n, d3o)
        vals = jnp.where(kcols == j, m, vals)
        idxs = jnp.where(kcols == j, outcol, idxs)
        return vals, idxs

    vals0 = jnp.zeros((ROWS, k), jnp.float32)
    idxs0 = jnp.zeros((ROWS, k), jnp.int32)
    vals, idxs = lax.fori_loop(0, k, step, (vals0, idxs0))
    bond_ref[0] = 1.0 / (1.0 + jnp.exp(-vals))
    gidx_ref[0] = idxs + (base + pl.program_id(0) * n)


def _topk_call(x, w_pad, bias_pad, scale, k, base):
    bsz, n, d = x.shape
    grid = (bsz, n // ROWS)
    return pl.pallas_call(
        functools.partial(_topk_body, n, k, base),
        grid=grid,
        in_specs=[
            pl.BlockSpec((1, 1), lambda b, i: (0, 0)),
            pl.BlockSpec((1, ROWS, d), lambda b, i: (b, i, 0)),
            pl.BlockSpec((1, n, d), lambda b, i: (b, 0, 0)),
            pl.BlockSpec((OPAD, d), lambda b, i: (0, 0)),
            pl.BlockSpec((1, OPAD), lambda b, i: (0, 0)),
        ],
        out_specs=[
            pl.BlockSpec((1, ROWS, k), lambda b, i: (b, i, 0)),
            pl.BlockSpec((1, ROWS, k), lambda b, i: (b, i, 0)),
        ],
        out_shape=[
            jax.ShapeDtypeStruct((bsz, n, k), jnp.float32),
            jax.ShapeDtypeStruct((bsz, n, k), jnp.int32),
        ],
        scratch_shapes=[pltpu.VMEM((ROWS, n), jnp.float32),
                        pltpu.VMEM((ROWS, n // PARTS), jnp.int32),
                        pltpu.VMEM((ROWS, n // PARTS), jnp.int32),
                        pltpu.VMEM((ROWS, n // PARTS), jnp.int32)],
    )(scale, x, x, w_pad, bias_pad)


def _gather_call(table, flat_idx):
    total, d = table.shape[0], table.shape[1]
    g = flat_idx.shape[0]
    info = plsc.get_sparse_core_info()
    nw = info.num_cores * info.num_subcores
    per_w = g // nw
    mesh = plsc.VectorSubcoreMesh(core_axis_name="c", subcore_axis_name="s")

    nchunks = per_w // CHUNK
    nbuf = 4

    @functools.partial(
        pl.kernel,
        out_type=jax.ShapeDtypeStruct((g, d), jnp.float32),
        mesh=mesh,
        scratch_types=[
            pltpu.VMEM((nbuf, CHUNK), jnp.int32),
            pltpu.VMEM((nbuf, CHUNK, d), jnp.float32),
            pltpu.SemaphoreType.DMA((nbuf,)),
            pltpu.SemaphoreType.DMA((nbuf,)),
            pltpu.SemaphoreType.DMA((nbuf,)),
        ],
    )
    def gather_k(table_hbm, idx_hbm, out_hbm, idx_v, rows_v, isem, gsem,
                 ssem):
        wid = lax.axis_index("s") * info.num_cores + lax.axis_index("c")
        base = wid * per_w

        def idx_cp(j, s):
            return pltpu.make_async_copy(
                idx_hbm.at[pl.ds(base + j * CHUNK, CHUNK)], idx_v.at[s],
                isem.at[s])

        def gather_cp(s):
            return pltpu.make_async_copy(table_hbm.at[idx_v.at[s]],
                                         rows_v.at[s], gsem.at[s])

        def scatter_cp(j, s):
            return pltpu.make_async_copy(
                rows_v.at[s], out_hbm.at[pl.ds(base + j * CHUNK, CHUNK)],
                ssem.at[s])

        idx_cp(0, 0).start()
        idx_cp(1, 1).start()

        def body(j, carry):
            s = j % nbuf
            idx_cp(j, s).wait()

            @pl.when(j >= nbuf)
            def _():
                scatter_cp(j - nbuf, s).wait()

            gather_cp(s).start()

            @pl.when(j >= 2)
            def _():
                gather_cp((j - 2) % nbuf).wait()
                scatter_cp(j - 2, (j - 2) % nbuf).start()

            @pl.when(j + 2 < nchunks)
            def _():
                idx_cp(j + 2, (j + 2) % nbuf).start()

            return carry

        lax.fori_loop(0, nchunks, body, 0)
        for t in (nchunks - 2, nchunks - 1):
            gather_cp(t % nbuf).wait()
            scatter_cp(t, t % nbuf).start()
        for t in range(nchunks - nbuf, nchunks):
            scatter_cp(t, t % nbuf).wait()

    return gather_k(table, flat_idx)


def kernel(x, edge_weight, node_weight, node_bias):
    bsz, n, d = x.shape
    k = 32
    o = node_weight.shape[0]
    w_pad = jnp.zeros((OPAD, d), jnp.float32).at[:o].set(node_weight)
    bias_pad = jnp.zeros((1, OPAD), jnp.float32).at[0, :o].set(node_bias)
    scale = jnp.exp(edge_weight).reshape(1, 1).astype(jnp.float32)

    table = x.reshape(bsz * n, d)
    bond, gidx = _topk_call(x, w_pad, bias_pad, scale, k, 0)
    rows = _gather_call(table, gidx.reshape(bsz * n * k))
    node_neighbor = rows.reshape(bsz, n, k, d)
    bond_neighbor = bond.reshape(bsz, n, k, 1)
    return node_neighbor, bond_neighbor


# speedup vs baseline: 1.1745x; 1.0510x over previous
import functools

import jax
import jax.numpy as jnp
from jax import lax
from jax.experimental import pallas as pl
from jax.experimental.pallas import tpu as pltpu
from jax.experimental.pallas import tpu_sc as plsc

ROWS = 512
OPAD = 16
CHUNK = 128


PARTS = 8
IDB = 3
SORT_NET = {
    4: ((0, 1), (2, 3), (0, 2), (1, 3), (1, 2)),
    8: ((0, 1), (2, 3), (4, 5), (6, 7),
        (0, 2), (1, 3), (4, 6), (5, 7),
        (1, 2), (5, 6),
        (0, 4), (1, 5), (2, 6), (3, 7),
        (2, 4), (3, 5),
        (1, 2), (3, 4), (5, 6)),
}


def _topk_body(n, k, base, scale_ref, x_rows_ref, x_full_ref, w_ref,
               bias_ref, bond_ref, gidx_ref, stack_ref, p2_ref, oc_ref,
               d_ref):
    i = pl.program_id(1)
    q = n // PARTS
    shq = q.bit_length() - 1
    w = w_ref[...]
    bias = bias_ref[...]

    def feat(v):
        t = lax.dot_general(v, w, (((1,), (1,)), ((), ())),
                            preferred_element_type=jnp.float32) + bias
        return jnp.where(t > 0, t, 0.01 * t)

    xt_rows = feat(x_rows_ref[0])
    xt_full = feat(x_full_ref[0])
    sim = lax.dot_general(xt_rows, xt_full, (((1,), (1,)), ((), ())),
                          preferred_element_type=jnp.float32)
    sim = sim * scale_ref[...]
    rows = i * ROWS + lax.broadcasted_iota(jnp.int32, (ROWS, n), 0)
    cols = lax.broadcasted_iota(jnp.int32, (ROWS, n), 1)
    sim = jnp.where(cols == rows, -jnp.inf, sim)

    parts = [lax.slice(sim, (0, m * q), (ROWS, (m + 1) * q))
             for m in range(PARTS)]
    ids = [jnp.full((ROWS, q), m, jnp.int32) for m in range(PARTS)]

    def cswap(a, ia, c, ic):
        sw = (c > a) | ((c == a) & (ic < ia))
        return (jnp.where(sw, c, a), jnp.where(sw, ic, ia),
                jnp.where(sw, a, c), jnp.where(sw, ia, ic))

    for x_, y_ in SORT_NET[PARTS]:
        parts[x_], ids[x_], parts[y_], ids[y_] = cswap(
            parts[x_], ids[x_], parts[y_], ids[y_])
    lane = lax.broadcasted_iota(jnp.int32, (ROWS, q), 1)
    for m in range(PARTS):
        stack_ref[:, m * q:(m + 1) * q] = parts[m]
    packed = ids[0]
    for m in range(1, PARTS):
        packed = packed | (ids[m] << (IDB * m))
    p2_ref[...] = packed
    oc_ref[...] = (ids[0] << shq) | lane
    d_ref[...] = jnp.zeros((ROWS, q), jnp.int32)

    kcols = lax.broadcasted_iota(jnp.int32, (ROWS, k), 1)

    def step(j, carry):
        vals, idxs = carry
        head = stack_ref[:, :q]
        ocv = oc_ref[...]
        m = jnp.max(head, axis=1, keepdims=True)
        eq = head == m
        outcol = jnp.min(jnp.where(eq, ocv, n), axis=1, keepdims=True)
        eqam = ocv == outcol
        d3o = d_ref[...]
        d3n = d3o + IDB
        nxt = jnp.full((ROWS, q), -jnp.inf)
        for lv in range(PARTS - 1, 0, -1):
            nxt = jnp.where(d3n == lv * IDB,
                            stack_ref[:, lv * q:(lv + 1) * q], nxt)
        newmem = (p2_ref[...] >> d3n) & (PARTS - 1)
        stack_ref[:, :q] = jnp.where(eqam, nxt, head)
        oc_ref[...] = jnp.where(eqam, (newmem << shq) | lane, ocv)
        d_ref[...] = jnp.where(eqam, d3n, d3o)
        vals = jnp.where(kcols == j, m, vals)
        idxs = jnp.where(kcols == j, outcol, idxs)
        return vals, idxs

    vals0 = jnp.zeros((ROWS, k), jnp.float32)
    idxs0 = jnp.zeros((ROWS, k), jnp.int32)
    vals, idxs = lax.fori_loop(0, k, step, (vals0, idxs0))
    bond_ref[0] = 1.0 / (1.0 + jnp.exp(-vals))
    gidx_ref[0] = idxs + (base + pl.program_id(0) * n)


def _topk_call(x, w_pad, bias_pad, scale, k, base):
    bsz, n, d = x.shape
    grid = (bsz, n // ROWS)
    return pl.pallas_call(
        functools.partial(_topk_body, n, k, base),
        grid=grid,
        in_specs=[
            pl.BlockSpec((1, 1), lambda b, i: (0, 0)),
            pl.BlockSpec((1, ROWS, d), lambda b, i: (b, i, 0)),
            pl.BlockSpec((1, n, d), lambda b, i: (b, 0, 0)),
            pl.BlockSpec((OPAD, d), lambda b, i: (0, 0)),
            pl.BlockSpec((1, OPAD), lambda b, i: (0, 0)),
        ],
        out_specs=[
            pl.BlockSpec((1, ROWS, k), lambda b, i: (b, i, 0)),
            pl.BlockSpec((1, ROWS, k), lambda b, i: (b, i, 0)),
        ],
        out_shape=[
            jax.ShapeDtypeStruct((bsz, n, k), jnp.float32),
            jax.ShapeDtypeStruct((bsz, n, k), jnp.int32),
        ],
        scratch_shapes=[pltpu.VMEM((ROWS, n), jnp.float32),
                        pltpu.VMEM((ROWS, n // PARTS), jnp.int32),
                        pltpu.VMEM((ROWS, n // PARTS), jnp.int32),
                        pltpu.VMEM((ROWS, n // PARTS), jnp.int32)],
    )(scale, x, x, w_pad, bias_pad)


def _gather_call(table, flat_idx):
    total, d = table.shape[0], table.shape[1]
    g = flat_idx.shape[0]
    info = plsc.get_sparse_core_info()
    nw = info.num_cores * info.num_subcores
    per_w = g // nw
    mesh = plsc.VectorSubcoreMesh(core_axis_name="c", subcore_axis_name="s")

    nchunks = per_w // CHUNK
    nbuf = 4

    @functools.partial(
        pl.kernel,
        out_type=jax.ShapeDtypeStruct((g, d), jnp.float32),
        mesh=mesh,
        scratch_types=[
            pltpu.VMEM((nbuf, CHUNK), jnp.int32),
            pltpu.VMEM((nbuf, CHUNK, d), jnp.float32),
            pltpu.SemaphoreType.DMA((nbuf,)),
            pltpu.SemaphoreType.DMA((nbuf,)),
            pltpu.SemaphoreType.DMA((nbuf,)),
        ],
    )
    def gather_k(table_hbm, idx_hbm, out_hbm, idx_v, rows_v, isem, gsem,
                 ssem):
        wid = lax.axis_index("s") * info.num_cores + lax.axis_index("c")
        base = wid * per_w

        def idx_cp(j, s):
            return pltpu.make_async_copy(
                idx_hbm.at[pl.ds(base + j * CHUNK, CHUNK)], idx_v.at[s],
                isem.at[s])

        def gather_cp(s):
            return pltpu.make_async_copy(table_hbm.at[idx_v.at[s]],
                                         rows_v.at[s], gsem.at[s])

        def scatter_cp(j, s):
            return pltpu.make_async_copy(
                rows_v.at[s], out_hbm.at[pl.ds(base + j * CHUNK, CHUNK)],
                ssem.at[s])

        idx_cp(0, 0).start()
        idx_cp(1, 1).start()

        def body(j, carry):
            s = j % nbuf
            idx_cp(j, s).wait()

            @pl.when(j >= nbuf)
            def _():
                scatter_cp(j - nbuf, s).wait()

            gather_cp(s).start()

            @pl.when(j >= 2)
            def _():
                gather_cp((j - 2) % nbuf).wait()
                scatter_cp(j - 2, (j - 2) % nbuf).start()

            @pl.when(j + 2 < nchunks)
            def _():
                idx_cp(j + 2, (j + 2) % nbuf).start()

            return carry

        lax.fori_loop(0, nchunks, body, 0)
        for t in (nchunks - 2, nchunks - 1):
            gather_cp(t % nbuf).wait()
            scatter_cp(t, t % nbuf).start()
        for t in range(nchunks - nbuf, nchunks):
            scatter_cp(t, t % nbuf).wait()

    return gather_k(table, flat_idx)


def kernel(x, edge_weight, node_weight, node_bias):
    bsz, n, d = x.shape
    k = 32
    o = node_weight.shape[0]
    w_pad = jnp.zeros((OPAD, d), jnp.float32).at[:o].set(node_weight)
    bias_pad = jnp.zeros((1, OPAD), jnp.float32).at[0, :o].set(node_bias)
    scale = jnp.exp(edge_weight).reshape(1, 1).astype(jnp.float32)

    table = x.reshape(bsz * n, d)
    bond, gidx = _topk_call(x, w_pad, bias_pad, scale, k, 0)
    rows = _gather_call(table, gidx.reshape(bsz * n * k))
    node_neighbor = rows.reshape(bsz, n, k, d)
    bond_neighbor = bond.reshape(bsz, n, k, 1)
    return node_neighbor, bond_neighbor


# speedup vs baseline: 1.1772x; 1.0023x over previous
import functools

import jax
import jax.numpy as jnp
from jax import lax
from jax.experimental import pallas as pl
from jax.experimental.pallas import tpu as pltpu
from jax.experimental.pallas import tpu_sc as plsc

ROWS = 1024
OPAD = 16
CHUNK = 128


PARTS = 8
IDB = 3
SORT_NET = {
    4: ((0, 1), (2, 3), (0, 2), (1, 3), (1, 2)),
    8: ((0, 1), (2, 3), (4, 5), (6, 7),
        (0, 2), (1, 3), (4, 6), (5, 7),
        (1, 2), (5, 6),
        (0, 4), (1, 5), (2, 6), (3, 7),
        (2, 4), (3, 5),
        (1, 2), (3, 4), (5, 6)),
}


def _topk_body(n, k, base, scale_ref, x_rows_ref, x_full_ref, w_ref,
               bias_ref, bond_ref, gidx_ref, stack_ref, p2_ref, oc_ref,
               d_ref):
    i = pl.program_id(1)
    q = n // PARTS
    shq = q.bit_length() - 1
    w = w_ref[...]
    bias = bias_ref[...]

    def feat(v):
        t = lax.dot_general(v, w, (((1,), (1,)), ((), ())),
                            preferred_element_type=jnp.float32) + bias
        return jnp.where(t > 0, t, 0.01 * t)

    xt_rows = feat(x_rows_ref[0])
    xt_full = feat(x_full_ref[0])
    sim = lax.dot_general(xt_rows, xt_full, (((1,), (1,)), ((), ())),
                          preferred_element_type=jnp.float32)
    sim = sim * scale_ref[...]
    rows = i * ROWS + lax.broadcasted_iota(jnp.int32, (ROWS, n), 0)
    cols = lax.broadcasted_iota(jnp.int32, (ROWS, n), 1)
    sim = jnp.where(cols == rows, -jnp.inf, sim)

    parts = [lax.slice(sim, (0, m * q), (ROWS, (m + 1) * q))
             for m in range(PARTS)]
    ids = [jnp.full((ROWS, q), m, jnp.int32) for m in range(PARTS)]

    def cswap(a, ia, c, ic):
        sw = (c > a) | ((c == a) & (ic < ia))
        return (jnp.where(sw, c, a), jnp.where(sw, ic, ia),
                jnp.where(sw, a, c), jnp.where(sw, ia, ic))

    for x_, y_ in SORT_NET[PARTS]:
        parts[x_], ids[x_], parts[y_], ids[y_] = cswap(
            parts[x_], ids[x_], parts[y_], ids[y_])
    lane = lax.broadcasted_iota(jnp.int32, (ROWS, q), 1)
    for m in range(PARTS):
        stack_ref[:, m * q:(m + 1) * q] = parts[m]
    packed = ids[0]
    for m in range(1, PARTS):
        packed = packed | (ids[m] << (IDB * m))
    p2_ref[...] = packed
    oc_ref[...] = (ids[0] << shq) | lane
    d_ref[...] = jnp.zeros((ROWS, q), jnp.int32)

    kcols = lax.broadcasted_iota(jnp.int32, (ROWS, k), 1)

    def step(j, carry):
        vals, idxs = carry
        head = stack_ref[:, :q]
        ocv = oc_ref[...]
        m = jnp.max(head, axis=1, keepdims=True)
        eq = head == m
        outcol = jnp.min(jnp.where(eq, ocv, n), axis=1, keepdims=True)
        eqam = ocv == outcol
        d3o = d_ref[...]
        d3n = d3o + IDB
        nxt = jnp.full((ROWS, q), -jnp.inf)
        for lv in range(PARTS - 1, 0, -1):
            nxt = jnp.where(d3n == lv * IDB,
                            stack_ref[:, lv * q:(lv + 1) * q], nxt)
        newmem = (p2_ref[...] >> d3n) & (PARTS - 1)
        stack_ref[:, :q] = jnp.where(eqam, nxt, head)
        oc_ref[...] = jnp.where(eqam, (newmem << shq) | lane, ocv)
        d_ref[...] = jnp.where(eqam, d3n, d3o)
        vals = jnp.where(kcols == j, m, vals)
        idxs = jnp.where(kcols == j, outcol, idxs)
        return vals, idxs

    vals0 = jnp.zeros((ROWS, k), jnp.float32)
    idxs0 = jnp.zeros((ROWS, k), jnp.int32)
    vals, idxs = lax.fori_loop(0, k, step, (vals0, idxs0))
    bond_ref[0] = 1.0 / (1.0 + jnp.exp(-vals))
    gidx_ref[0] = idxs + (base + pl.program_id(0) * n)


def _topk_call(x, w_pad, bias_pad, scale, k, base):
    bsz, n, d = x.shape
    grid = (bsz, n // ROWS)
    return pl.pallas_call(
        functools.partial(_topk_body, n, k, base),
        grid=grid,
        in_specs=[
            pl.BlockSpec((1, 1), lambda b, i: (0, 0)),
            pl.BlockSpec((1, ROWS, d), lambda b, i: (b, i, 0)),
            pl.BlockSpec((1, n, d), lambda b, i: (b, 0, 0)),
            pl.BlockSpec((OPAD, d), lambda b, i: (0, 0)),
            pl.BlockSpec((1, OPAD), lambda b, i: (0, 0)),
        ],
        out_specs=[
            pl.BlockSpec((1, ROWS, k), lambda b, i: (b, i, 0)),
            pl.BlockSpec((1, ROWS, k), lambda b, i: (b, i, 0)),
        ],
        out_shape=[
            jax.ShapeDtypeStruct((bsz, n, k), jnp.float32),
            jax.ShapeDtypeStruct((bsz, n, k), jnp.int32),
        ],
        scratch_shapes=[pltpu.VMEM((ROWS, n), jnp.float32),
                        pltpu.VMEM((ROWS, n // PARTS), jnp.int32),
                        pltpu.VMEM((ROWS, n // PARTS), jnp.int32),
                        pltpu.VMEM((ROWS, n // PARTS), jnp.int32)],
    )(scale, x, x, w_pad, bias_pad)


def _gather_call(table, flat_idx):
    total, d = table.shape[0], table.shape[1]
    g = flat_idx.shape[0]
    info = plsc.get_sparse_core_info()
    nw = info.num_cores * info.num_subcores
    per_w = g // nw
    mesh = plsc.VectorSubcoreMesh(core_axis_name="c", subcore_axis_name="s")

    nchunks = per_w // CHUNK
    nbuf = 4

    @functools.partial(
        pl.kernel,
        out_type=jax.ShapeDtypeStruct((g, d), jnp.float32),
        mesh=mesh,
        scratch_types=[
            pltpu.VMEM((nbuf, CHUNK), jnp.int32),
            pltpu.VMEM((nbuf, CHUNK, d), jnp.float32),
            pltpu.SemaphoreType.DMA((nbuf,)),
            pltpu.SemaphoreType.DMA((nbuf,)),
            pltpu.SemaphoreType.DMA((nbuf,)),
        ],
    )
    def gather_k(table_hbm, idx_hbm, out_hbm, idx_v, rows_v, isem, gsem,
                 ssem):
        wid = lax.axis_index("s") * info.num_cores + lax.axis_index("c")
        base = wid * per_w

        def idx_cp(j, s):
            return pltpu.make_async_copy(
                idx_hbm.at[pl.ds(base + j * CHUNK, CHUNK)], idx_v.at[s],
                isem.at[s])

        def gather_cp(s):
            return pltpu.make_async_copy(table_hbm.at[idx_v.at[s]],
                                         rows_v.at[s], gsem.at[s])

        def scatter_cp(j, s):
            return pltpu.make_async_copy(
                rows_v.at[s], out_hbm.at[pl.ds(base + j * CHUNK, CHUNK)],
                ssem.at[s])

        idx_cp(0, 0).start()
        idx_cp(1, 1).start()

        def body(j, carry):
            s = j % nbuf
            idx_cp(j, s).wait()

            @pl.when(j >= nbuf)
            def _():
                scatter_cp(j - nbuf, s).wait()

            gather_cp(s).start()

            @pl.when(j >= 2)
            def _():
                gather_cp((j - 2) % nbuf).wait()
                scatter_cp(j - 2, (j - 2) % nbuf).start()

            @pl.when(j + 2 < nchunks)
            def _():
                idx_cp(j + 2, (j + 2) % nbuf).start()

            return carry

        lax.fori_loop(0, nchunks, body, 0)
        for t in (nchunks - 2, nchunks - 1):
            gather_cp(t % nbuf).wait()
            scatter_cp(t, t % nbuf).start()
        for t in range(nchunks - nbuf, nchunks):
            scatter_cp(t, t % nbuf).wait()

    return gather_k(table, flat_idx)


def kernel(x, edge_weight, node_weight, node_bias):
    bsz, n, d = x.shape
    k = 32
    o = node_weight.shape[0]
    w_pad = jnp.zeros((OPAD, d), jnp.float32).at[:o].set(node_weight)
    bias_pad = jnp.zeros((1, OPAD), jnp.float32).at[0, :o].set(node_bias)
    scale = jnp.exp(edge_weight).reshape(1, 1).astype(jnp.float32)

    table = x.reshape(bsz * n, d)
    bond, gidx = _topk_call(x, w_pad, bias_pad, scale, k, 0)
    rows = _gather_call(table, gidx.reshape(bsz * n * k))
    node_neighbor = rows.reshape(bsz, n, k, d)
    bond_neighbor = bond.reshape(bsz, n, k, 1)
    return node_neighbor, bond_neighbor
